# baseline (speedup 1.0000x reference)
import jax
import jax.numpy as jnp
from jax.experimental import pallas as pl
from jax.experimental.pallas import tpu as pltpu

_TN = 256


def _vq_block(xt_ref, x2_ref, cb_ref, cbm2_ref, c2_ref,
              oh_ref, xhat_ref, idx_ref):
    G = cb_ref.shape[0]
    CB = cb_ref.shape[1]
    TN = x2_ref.shape[0]
    idxs = []
    for g in range(G):
        xg = xt_ref[g]
        prod = jax.lax.dot_general(
            xg, cbm2_ref[g], (((1,), (1,)), ((), ())),
            preferred_element_type=jnp.float32)
        bias = x2_ref[:, g:g + 1] + c2_ref[g:g + 1, :]
        dist = bias + prod
        idx = jnp.argmin(dist, axis=1)
        idxs.append(idx[:, None])
    idx2 = jnp.concatenate(idxs, axis=1)
    iota3 = jax.lax.broadcasted_iota(jnp.int32, (TN, G, CB), 2)
    oh3 = (iota3 == idx2[:, :, None]).astype(jnp.float32)
    oh_ref[:, :, :] = oh3
    idx_ref[:, :] = idx2
    for g in range(G):
        xhat_ref[:, g, :] = jnp.dot(
            oh3[:, g, :], cb_ref[g], preferred_element_type=jnp.float32)


def kernel(x, code_book):
    n, G, dim = x.shape
    CB = code_book.shape[1]
    xt = x.transpose(1, 0, 2)
    x2 = jnp.sum(x * x, axis=-1)
    c2 = jnp.sum(code_book * code_book, axis=-1)
    cbm2 = -2.0 * code_book
    one_hot, x_hat, index = pl.pallas_call(
        _vq_block,
        grid=(n // _TN,),
        in_specs=[
            pl.BlockSpec((G, _TN, dim), lambda i: (0, i, 0)),
            pl.BlockSpec((_TN, G), lambda i: (i, 0)),
            pl.BlockSpec((G, CB, dim), lambda i: (0, 0, 0)),
            pl.BlockSpec((G, CB, dim), lambda i: (0, 0, 0)),
            pl.BlockSpec((G, CB), lambda i: (0, 0)),
        ],
        out_specs=[
            pl.BlockSpec((_TN, G, CB), lambda i: (i, 0, 0)),
            pl.BlockSpec((_TN, G, dim), lambda i: (i, 0, 0)),
            pl.BlockSpec((_TN, G), lambda i: (i, 0)),
        ],
        out_shape=[
            jax.ShapeDtypeStruct((n, G, CB), jnp.float32),
            jax.ShapeDtypeStruct((n, G, dim), jnp.float32),
            jax.ShapeDtypeStruct((n, G), jnp.int32),
        ],
        compiler_params=pltpu.CompilerParams(
            dimension_semantics=("parallel",)),
    )(xt, x2, code_book, cbm2, c2)
    return (x_hat, one_hot, index[..., None])


# speedup vs baseline: 1.3407x; 1.3407x over previous
import jax
import jax.numpy as jnp
from jax.experimental import pallas as pl
from jax.experimental.pallas import tpu as pltpu

_TN = 256
_LC = 128


def _vq_block(xt_ref, x2_ref, cb_ref, cbm2_ref, c2_ref,
              oh_ref, xhat_ref, idx_ref):
    G = cb_ref.shape[0]
    CB = cb_ref.shape[1]
    TN = x2_ref.shape[0]
    nc = CB // _LC
    iota_c = jax.lax.broadcasted_iota(jnp.int32, (TN, _LC), 1)
    iota_f = jax.lax.broadcasted_iota(jnp.int32, (TN, CB), 1)
    for g in range(G):
        xg = xt_ref[g]
        prod = jax.lax.dot_general(
            xg, cbm2_ref[g], (((1,), (1,)), ((), ())),
            preferred_element_type=jnp.float32)
        x2g = x2_ref[:, g:g + 1]
        rv = (x2g + c2_ref[g:g + 1, 0:_LC]) + prod[:, 0:_LC]
        ri = iota_c
        for c in range(1, nc):
            lo = c * _LC
            d = (x2g + c2_ref[g:g + 1, lo:lo + _LC]) + prod[:, lo:lo + _LC]
            upd = d < rv
            ri = jnp.where(upd, iota_c + lo, ri)
            rv = jnp.where(upd, d, rv)
        m = jnp.min(rv, axis=1, keepdims=True)
        cand = jnp.where(rv == m, ri, CB)
        idx = jnp.min(cand, axis=1, keepdims=True)
        oh = (iota_f == idx).astype(jnp.float32)
        oh_ref[:, g, :] = oh
        xhat_ref[:, g, :] = jnp.dot(
            oh, cb_ref[g], preferred_element_type=jnp.float32)
        idx_ref[:, g:g + 1] = idx


def kernel(x, code_book):
    n, G, dim = x.shape
    CB = code_book.shape[1]
    xt = x.transpose(1, 0, 2)
    x2 = jnp.sum(x * x, axis=-1)
    c2 = jnp.sum(code_book * code_book, axis=-1)
    cbm2 = -2.0 * code_book
    one_hot, x_hat, index = pl.pallas_call(
        _vq_block,
        grid=(n // _TN,),
        in_specs=[
            pl.BlockSpec((G, _TN, dim), lambda i: (0, i, 0)),
            pl.BlockSpec((_TN, G), lambda i: (i, 0)),
            pl.BlockSpec((G, CB, dim), lambda i: (0, 0, 0)),
            pl.BlockSpec((G, CB, dim), lambda i: (0, 0, 0)),
            pl.BlockSpec((G, CB), lambda i: (0, 0)),
        ],
        out_specs=[
            pl.BlockSpec((_TN, G, CB), lambda i: (i, 0, 0)),
            pl.BlockSpec((_TN, G, dim), lambda i: (i, 0, 0)),
            pl.BlockSpec((_TN, G), lambda i: (i, 0)),
        ],
        out_shape=[
            jax.ShapeDtypeStruct((n, G, CB), jnp.float32),
            jax.ShapeDtypeStruct((n, G, dim), jnp.float32),
            jax.ShapeDtypeStruct((n, G), jnp.int32),
        ],
        compiler_params=pltpu.CompilerParams(
            dimension_semantics=("parallel",)),
    )(xt, x2, code_book, cbm2, c2)
    return (x_hat, one_hot, index[..., None])
